# Initial kernel scaffold; baseline (speedup 1.0000x reference)
import jax
import jax.numpy as jnp
from jax.experimental import pallas as pl


def kernel(point_seq):
    raise NotImplementedError("write your pallas kernel here")


# speedup vs baseline: 24.2758x; 24.2758x over previous
import functools

import jax
import jax.numpy as jnp
from jax import lax
from jax.experimental import pallas as pl
from jax.experimental.pallas import tpu as pltpu, tpu_sc as plsc

_K = 16


def _knn_body(x_ref, xT_ref, px_ref, py_ref, pz_ref, idx_ref, dist_ref,
              outx_ref, xc_ref, d_ref, *, n, k):
    i = pl.program_id(1)
    xn = x_ref[0, 0]
    xnT = xT_ref[0, 0]
    rows = (px_ref[0, 0], py_ref[0, 0], pz_ref[0, 0])

    @pl.when(i == 0)
    def _():
        xc_ref[...] = xn

    xc = xc_ref[...]
    sq_c = jnp.sum(xc * xc, axis=1, keepdims=True)
    sq_n = jnp.sum(xnT * xnT, axis=0, keepdims=True)
    dot = lax.dot_general(xc, xnT, (((1,), (0,)), ((), ())),
                          preferred_element_type=jnp.float32)
    d_ref[...] = sq_c + sq_n - 2.0 * dot

    iota = lax.broadcasted_iota(jnp.int32, (n, n), 1)
    for j in range(k):
        d = d_ref[...]
        m = jnp.min(d, axis=1, keepdims=True)
        cand = jnp.where(d == m, iota, n)
        idxv = jnp.min(cand, axis=1, keepdims=True)
        sel = iota == idxv
        dist_ref[0, 0, :, j:j + 1] = m
        idx_ref[0, 0, :, j:j + 1] = idxv
        d_ref[...] = jnp.where(sel, jnp.inf, d)
        if j == 0:
            xc_new = jnp.concatenate(
                [jnp.min(jnp.where(sel, rows[c], jnp.inf),
                         axis=1, keepdims=True) for c in range(3)], axis=1)
            xc_ref[...] = xc_new
            outx_ref[0, 0] = xc_new


def _tc_knn(ps, interpret=False):
    b, t, n, d = ps.shape
    k = _K
    psT = jnp.swapaxes(ps, 2, 3)
    coord = [psT[:, :, c:c + 1, :] for c in range(3)]
    return pl.pallas_call(
        functools.partial(_knn_body, n=n, k=k),
        grid=(b, t),
        in_specs=[
            pl.BlockSpec((1, 1, n, d), lambda bi, ti: (bi, ti, 0, 0)),
            pl.BlockSpec((1, 1, d, n), lambda bi, ti: (bi, ti, 0, 0)),
            pl.BlockSpec((1, 1, 1, n), lambda bi, ti: (bi, ti, 0, 0)),
            pl.BlockSpec((1, 1, 1, n), lambda bi, ti: (bi, ti, 0, 0)),
            pl.BlockSpec((1, 1, 1, n), lambda bi, ti: (bi, ti, 0, 0)),
        ],
        out_specs=[
            pl.BlockSpec((1, 1, n, k), lambda bi, ti: (bi, ti, 0, 0)),
            pl.BlockSpec((1, 1, n, k), lambda bi, ti: (bi, ti, 0, 0)),
            pl.BlockSpec((1, 1, n, d), lambda bi, ti: (bi, ti, 0, 0)),
        ],
        out_shape=[
            jax.ShapeDtypeStruct((b, t, n, k), jnp.int32),
            jax.ShapeDtypeStruct((b, t, n, k), jnp.float32),
            jax.ShapeDtypeStruct((b, t, n, d), jnp.float32),
        ],
        scratch_shapes=[
            pltpu.VMEM((n, d), jnp.float32),
            pltpu.VMEM((n, n), jnp.float32),
        ],
        compiler_params=pltpu.CompilerParams(
            dimension_semantics=("arbitrary", "arbitrary")),
        interpret=interpret,
    )(ps, psT, *coord)


def _sc_gather_body(ps_hbm, idx_hbm, pp_hbm, norm_hbm, feat_hbm,
                    f0_v, f1_v, idx_v, anc_v, pp_v, nrm_v, fts_v, sem):
    T, QB = 16, 256
    nc = 2
    wid = lax.axis_index("s") * nc + lax.axis_index("c")
    bb = wid // 4
    q0 = (wid % 4) * QB

    iota16 = lax.broadcasted_iota(jnp.int32, (16,), 0)
    z16 = jnp.zeros((16,), jnp.int32)
    i3 = iota16 * 3
    i6 = iota16 * 6

    fbufs = (f0_v, f1_v)
    for t in range(T):
        cur = fbufs[t % 2]
        prev = fbufs[(t - 1) % 2] if t > 0 else fbufs[0]
        pltpu.sync_copy(ps_hbm.at[bb, t], cur)
        pltpu.sync_copy(idx_hbm.at[bb, t, pl.ds(q0 * 16, QB * 16)], idx_v)

        def row(q, carry, t=t, cur=cur, prev=prev):
            idxv = idx_v[pl.ds(q * 16, 16)]
            idx3 = idxv * 3
            p = [plsc.load_gather(cur, [idx3 + c]) for c in range(3)]
            pf = [plsc.load_gather(prev, [idx3 + c]) for c in range(3)]
            if t == 0:
                aidx = plsc.load_gather(idx_v, [z16 + q * 16])
                a3 = aidx * 3
                a = [plsc.load_gather(cur, [a3 + c]) for c in range(3)]
                for c in range(3):
                    anc_v[pl.ds((q * 3 + c) * 16, 16)] = a[c]
            else:
                a = [anc_v[pl.ds((q * 3 + c) * 16, 16)] for c in range(3)]
            o3 = i3 + q * 48
            o6 = i6 + q * 96
            for c in range(3):
                nv = p[c] - a[c]
                plsc.store_scatter(pp_v, [o3 + c], p[c])
                plsc.store_scatter(nrm_v, [o3 + c], nv)
                plsc.store_scatter(fts_v, [o6 + c], pf[c])
                plsc.store_scatter(fts_v, [o6 + (3 + c)], nv)
            return carry

        lax.fori_loop(0, QB, row, 0)
        pltpu.sync_copy(pp_v, pp_hbm.at[bb, t, pl.ds(q0 * 48, QB * 48)])
        pltpu.sync_copy(nrm_v, norm_hbm.at[bb, t, pl.ds(q0 * 48, QB * 48)])
        pltpu.sync_copy(fts_v, feat_hbm.at[bb, t, pl.ds(q0 * 96, QB * 96)])


def _sc_gather(ps, idxs):
    B, T, N, D = ps.shape
    K, QB = _K, 256
    mesh = plsc.VectorSubcoreMesh(core_axis_name="c", subcore_axis_name="s")
    fn = pl.kernel(
        _sc_gather_body,
        mesh=mesh,
        compiler_params=pltpu.CompilerParams(needs_layout_passes=False),
        out_type=[
            jax.ShapeDtypeStruct((B, T, N * K * 3), jnp.float32),
            jax.ShapeDtypeStruct((B, T, N * K * 3), jnp.float32),
            jax.ShapeDtypeStruct((B, T, N * K * 6), jnp.float32),
        ],
        scratch_types=[
            pltpu.VMEM((N * D,), jnp.float32),
            pltpu.VMEM((N * D,), jnp.float32),
            pltpu.VMEM((QB * K,), jnp.int32),
            pltpu.VMEM((QB * 3 * 16,), jnp.float32),
            pltpu.VMEM((QB * K * 3,), jnp.float32),
            pltpu.VMEM((QB * K * 3,), jnp.float32),
            pltpu.VMEM((QB * K * 6,), jnp.float32),
            pltpu.SemaphoreType.DMA,
        ],
    )
    pp, nrm, fts = fn(ps.reshape(B, T, N * D), idxs.reshape(B, T, N * K))
    return (pp.reshape(B, T, N, K, 3), nrm.reshape(B, T, N, K, 3),
            fts.reshape(B, T, N, K, 6))


def kernel(point_seq):
    idxs, dists, out_x = _tc_knn(point_seq)
    pp, nrm, fts = _sc_gather(point_seq, idxs)
    return (idxs, dists, idxs, pp, fts, nrm, out_x)
